# Initial kernel scaffold; baseline (speedup 1.0000x reference)
import jax
import jax.numpy as jnp
from jax.experimental import pallas as pl


def kernel(x, edge_index, W_cheb, b_cheb, W_lin, b_lin):
    raise NotImplementedError("write your pallas kernel here")


# speedup vs baseline: 11.1705x; 11.1705x over previous
import functools

import jax
import jax.numpy as jnp
from jax import lax
from jax.experimental import pallas as pl
from jax.experimental.pallas import tpu as pltpu
from jax.experimental.pallas import tpu_sc as plsc

N = 10000
E = 160000
F = 256
HF = 128
NCLASS = 40

NC = 2
NS = 16
L = 16

EPT = E // NS
C = 80
NCH = EPT // C
NPAD = 10240
NPT = NPAD // NS
XROWS = 2 * N + 16
X2ROWS = 2 * NPAD

_MESH = plsc.VectorSubcoreMesh(
    core_axis_name="c", subcore_axis_name="s", num_cores=NC, num_subcores=NS
)

_DEG_ROWS = 128
NBUF = 4


def _zero_vmem2d(ref, rows, cols):
    z = jnp.zeros((L,), ref.dtype)

    def body(j, _):
        for v in range(cols // L):
            ref[j, pl.ds(v * L, L)] = z
        return 0

    lax.fori_loop(0, rows, body, 0)


@functools.partial(
    pl.kernel,
    out_type=jax.ShapeDtypeStruct((_DEG_ROWS * HF,), jnp.float32),
    mesh=_MESH,
    compiler_params=pltpu.CompilerParams(needs_layout_passes=False),
    scratch_types=[
        pltpu.VMEM((NCH, C), jnp.int32),
        pltpu.VMEM((NCH, C), jnp.int32),
        pltpu.VMEM((_DEG_ROWS, HF), jnp.float32),
        pltpu.VMEM((_DEG_ROWS // NS, HF), jnp.float32),
        pltpu.VMEM((_DEG_ROWS,), jnp.int32),
        pltpu.VMEM((_DEG_ROWS // NS * HF,), jnp.float32),
        pltpu.VMEM_SHARED((_DEG_ROWS, HF), jnp.float32),
    ],
)
def _sc_prep(src_hbm, dst_hbm, dinv_hbm, srcb, dstb, hist, vbuf, rowidx, vout,
             deg_sh):
    c = lax.axis_index("c")
    s = lax.axis_index("s")
    rpt = _DEG_ROWS // NS

    _zero_vmem2d(vbuf, rpt, HF)
    pltpu.sync_copy(vbuf, deg_sh.at[pl.ds(s * rpt, rpt)])
    _zero_vmem2d(hist, _DEG_ROWS, HF)
    lane = lax.iota(jnp.int32, L)
    for k in range(_DEG_ROWS // L):
        rowidx[pl.ds(k * L, L)] = lane + k * L

    pltpu.sync_copy(src_hbm.at[s], srcb)
    pltpu.sync_copy(dst_hbm.at[s], dstb)
    ones = jnp.full((L,), 1.0, jnp.float32)

    def hist_body(j, _):
        for v in range(C // L):
            sv = srcb[j, pl.ds(v * L, L)]
            dv = dstb[j, pl.ds(v * L, L)]
            ne = sv != dv
            row = lax.shift_right_logical(sv, 7)
            col = lax.bitwise_and(sv, 127)
            for ln in range(L):
                m = lax.bitwise_and(ne, lane == ln)
                plsc.addupdate_scatter(hist, [row, col], ones, mask=m)
        return 0

    lax.fori_loop(0, NCH, hist_body, 0)

    plsc.subcore_barrier()
    pltpu.sync_copy(hist, deg_sh.at[rowidx], add=True)
    plsc.subcore_barrier()

    pltpu.sync_copy(deg_sh.at[pl.ds(s * rpt, rpt)], vbuf)
    magic = jnp.full((L,), 0x5F3759DF, jnp.int32)
    for r in range(rpt):
        for v in range(HF // L):
            d = vbuf[r, pl.ds(v * L, L)]
            di = plsc.bitcast(d, jnp.int32)
            y = plsc.bitcast(magic - lax.shift_right_logical(di, 1), jnp.float32)
            hd = 0.5 * d
            for _ in range(3):
                t = (hd * y) * y
                y = y * (1.5 - t)
            y = jnp.where(d > 0.0, y, 0.0)
            vout[pl.ds(r * HF + v * L, L)] = y
    @pl.when(c == 0)
    def _():
        pltpu.sync_copy(vout, dinv_hbm.at[pl.ds(s * rpt * HF, rpt * HF)])


@functools.partial(
    pl.kernel,
    out_type=[
        jax.ShapeDtypeStruct((NC, NPAD, HF), jnp.float32),
        jax.ShapeDtypeStruct((NC, NPAD, HF), jnp.float32),
        jax.ShapeDtypeStruct((X2ROWS, HF), jnp.float32),
    ],
    mesh=_MESH,
    compiler_params=pltpu.CompilerParams(needs_layout_passes=False),
    scratch_types=[
        [pltpu.VMEM((2, C), jnp.int32) for _ in range(NBUF)],
        [pltpu.VMEM((C,), jnp.int32) for _ in range(NBUF)],
        [pltpu.VMEM((C, HF), jnp.float32) for _ in range(NBUF)],
        pltpu.VMEM((16, HF), jnp.float32),
        pltpu.VMEM((C,), jnp.float32),
        [pltpu.SemaphoreType.DMA for _ in range(NBUF)],
        [pltpu.SemaphoreType.DMA for _ in range(NBUF)],
        pltpu.VMEM_SHARED((NPAD, HF), jnp.float32),
    ],
)
def _sc_prop2x(xf_hbm, ep_hbm, dinv_hbm, s1_hbm, s2_hbm, xf2_hbm,
               ebufs, gbufs, rbufs, zbuf, dinvq, gsems, ssems, acc):
    c = lax.axis_index("c")
    s = lax.axis_index("s")

    def zero_my_acc_rows():
        for k in range(NPT // 16):
            pltpu.sync_copy(zbuf, acc.at[pl.ds(s * NPT + k * 16, 16)])

    def stage_chunk(j, p, off, zrow):
        pltpu.sync_copy(ep_hbm.at[s, j], ebufs[p])
        for v in range(C // L):
            sv = ebufs[p][0, pl.ds(v * L, L)]
            dv = ebufs[p][1, pl.ds(v * L, L)]
            gbufs[p][pl.ds(v * L, L)] = jnp.where(sv != dv, sv, zrow) + off

    def run_pass(table_hbm, off, zrow):
        def issue_gather(j, p):
            pltpu.async_copy(table_hbm.at[gbufs[p]], rbufs[p], gsems[p])

        def wait_gather(j, p):
            pltpu.make_async_copy(table_hbm.at[gbufs[p]], rbufs[p], gsems[p]).wait()

        def issue_scatter(j, p):
            pltpu.async_copy(rbufs[p], acc.at[ebufs[p].at[1]], ssems[p], add=True)

        def wait_scatter(j, p):
            pltpu.make_async_copy(rbufs[p], acc.at[ebufs[p].at[1]], ssems[p]).wait()

        for p in range(NBUF):
            stage_chunk(p, p, off, zrow)
            issue_gather(p, p)

        def body(i, _):
            jb = i * NBUF
            for p in range(NBUF):
                j = jb + p
                wait_gather(j, p)
                issue_scatter(j, p)
                wait_scatter(j, p)
                stage_chunk(j + NBUF, p, off, zrow)
                issue_gather(j + NBUF, p)
            return 0

        ngroups = NCH // NBUF - 1
        lax.fori_loop(0, ngroups, body, 0)

        for j in range(ngroups * NBUF, NCH):
            p = j % NBUF
            wait_gather(j, p)
            issue_scatter(j, p)
            wait_scatter(j, p)
            if j + NBUF < NCH:
                stage_chunk(j + NBUF, p, off, zrow)
                issue_gather(j + NBUF, p)

    one16 = jnp.full((L,), 1, jnp.int32)
    _zero_vmem2d(zbuf, 16, HF)

    zero_my_acc_rows()
    plsc.subcore_barrier()
    run_pass(xf_hbm, c * N, jnp.int32(2 * N))
    plsc.subcore_barrier()

    for k in range(NPT // C):
        base = s * NPT + k * C
        pltpu.sync_copy(acc.at[pl.ds(base, C)], rbufs[0])
        pltpu.sync_copy(rbufs[0], s1_hbm.at[c, pl.ds(base, C)])
        pltpu.sync_copy(dinv_hbm.at[pl.ds(base, C)], dinvq)

        def row_body(r, _):
            dv = plsc.load_gather(dinvq, [one16 * r])
            gg = -dv * dv
            for v in range(HF // L):
                rbufs[1][r, pl.ds(v * L, L)] = gg * rbufs[0][r, pl.ds(v * L, L)]
            return 0

        lax.fori_loop(0, C, row_body, 0)
        pltpu.sync_copy(rbufs[1], xf2_hbm.at[pl.ds(c * NPAD + base, C)])

    zero_my_acc_rows()
    plsc.subcore_barrier()
    run_pass(xf2_hbm, c * NPAD, jnp.int32(N))
    plsc.subcore_barrier()

    pltpu.sync_copy(acc.at[pl.ds(s * NPT, NPT)], s2_hbm.at[c, pl.ds(s * NPT, NPT)])


_BR = 1000


def _b1_body(x_ref, d_ref, o_ref):
    i = pl.program_id(0)
    nb = N // _BR
    half = jnp.minimum(i // nb, 1)
    val = d_ref[...] * x_ref[:, pl.ds(half * HF, HF)]
    o_ref[...] = jnp.where(i < 2 * nb, val, 0.0)


def _tc_scale_flat(x, dinv2):
    grid = 2 * (N // _BR) + 1
    nb = N // _BR
    return pl.pallas_call(
        _b1_body,
        grid=(grid,),
        in_specs=[
            pl.BlockSpec((_BR, F), lambda i: (i % nb, 0)),
            pl.BlockSpec((_BR, 1), lambda i: (i % nb, 0)),
        ],
        out_specs=pl.BlockSpec((_BR, HF), lambda i: (i, 0)),
        out_shape=jax.ShapeDtypeStruct((XROWS, HF), jnp.float32),
    )(x, dinv2)


def _final_body(x_ref, s1a, s1b, s2a, s2b, d_ref, a0, a1, a2, bc, wl, bl, o_ref):
    d = d_ref[...]
    q1 = d * jnp.concatenate([s1a[0], s1b[0]], axis=1)
    q2 = d * jnp.concatenate([s2a[0], s2b[0]], axis=1)
    u = jnp.dot(x_ref[...], a0[...], preferred_element_type=jnp.float32)
    u += jnp.dot(q1, a1[...], preferred_element_type=jnp.float32)
    u += jnp.dot(q2, a2[...], preferred_element_type=jnp.float32)
    u += bc[...]
    h = jnp.maximum(u, 0.0)
    o_ref[...] = jnp.dot(h, wl[...], preferred_element_type=jnp.float32) + bl[...]


def _tc_final(x, s1, s2, dinv2, A0, A1, A2, bc, Wl, bl):
    nb = N // _BR
    wspec = pl.BlockSpec((F, F), lambda i: (0, 0))
    h0 = lambda i: (0, i, 0)
    h1 = lambda i: (1, i, 0)
    return pl.pallas_call(
        _final_body,
        grid=(nb,),
        in_specs=[
            pl.BlockSpec((_BR, F), lambda i: (i, 0)),
            pl.BlockSpec((1, _BR, HF), h0),
            pl.BlockSpec((1, _BR, HF), h1),
            pl.BlockSpec((1, _BR, HF), h0),
            pl.BlockSpec((1, _BR, HF), h1),
            pl.BlockSpec((_BR, 1), lambda i: (i, 0)),
            wspec, wspec, wspec,
            pl.BlockSpec((1, F), lambda i: (0, 0)),
            pl.BlockSpec((F, NCLASS), lambda i: (0, 0)),
            pl.BlockSpec((1, NCLASS), lambda i: (0, 0)),
        ],
        out_specs=pl.BlockSpec((_BR, NCLASS), lambda i: (i, 0)),
        out_shape=jax.ShapeDtypeStruct((N, NCLASS), jnp.float32),
    )(x, s1, s1, s2, s2, dinv2, A0, A1, A2, bc, Wl, bl)


def kernel(x, edge_index, W_cheb, b_cheb, W_lin, b_lin):
    ei = edge_index.astype(jnp.int32)
    srcb = ei[0].reshape(NS, NCH, C)
    dstb = ei[1].reshape(NS, NCH, C)
    epack = jnp.stack([srcb, dstb], axis=2)

    dinv1d = _sc_prep(srcb, dstb)
    dinv2 = dinv1d[:N].reshape(N, 1)

    x1f = _tc_scale_flat(x, dinv2)
    s1, s2, _ = _sc_prop2x(x1f, epack, dinv1d)

    A0 = W_cheb[0] - W_cheb[2]
    A1 = -W_cheb[1]
    A2 = -2.0 * W_cheb[2]
    return _tc_final(x, s1, s2, dinv2, A0, A1, A2,
                     b_cheb.reshape(1, F), W_lin, b_lin.reshape(1, NCLASS))


# speedup vs baseline: 12.0737x; 1.0809x over previous
import functools

import jax
import jax.numpy as jnp
from jax import lax
from jax.experimental import pallas as pl
from jax.experimental.pallas import tpu as pltpu
from jax.experimental.pallas import tpu_sc as plsc

N = 10000
E = 160000
F = 256
HF = 128
NCLASS = 40

NC = 2
NS = 16
L = 16

EPT = E // NS
C = 80
NCH = EPT // C
NPAD = 10240
NPT = NPAD // NS
XROWS = 2 * N + 16
X2ROWS = 2 * NPAD

_MESH = plsc.VectorSubcoreMesh(
    core_axis_name="c", subcore_axis_name="s", num_cores=NC, num_subcores=NS
)

_DEG_ROWS = 128
NBUF = 4


def _zero_vmem2d(ref, rows, cols):
    z = jnp.zeros((L,), ref.dtype)

    def body(j, _):
        for v in range(cols // L):
            ref[j, pl.ds(v * L, L)] = z
        return 0

    lax.fori_loop(0, rows, body, 0)


@functools.partial(
    pl.kernel,
    out_type=jax.ShapeDtypeStruct((_DEG_ROWS * HF,), jnp.float32),
    mesh=_MESH,
    compiler_params=pltpu.CompilerParams(needs_layout_passes=False),
    scratch_types=[
        pltpu.VMEM((NCH, C), jnp.int32),
        pltpu.VMEM((NCH, C), jnp.int32),
        pltpu.VMEM((_DEG_ROWS, HF), jnp.float32),
        pltpu.VMEM((_DEG_ROWS // NS, HF), jnp.float32),
        pltpu.VMEM((_DEG_ROWS,), jnp.int32),
        pltpu.VMEM((_DEG_ROWS // NS * HF,), jnp.float32),
        pltpu.VMEM_SHARED((_DEG_ROWS, HF), jnp.float32),
    ],
)
def _sc_prep(src_hbm, dst_hbm, dinv_hbm, srcb, dstb, hist, vbuf, rowidx, vout,
             deg_sh):
    c = lax.axis_index("c")
    s = lax.axis_index("s")
    rpt = _DEG_ROWS // NS

    _zero_vmem2d(vbuf, rpt, HF)
    pltpu.sync_copy(vbuf, deg_sh.at[pl.ds(s * rpt, rpt)])
    _zero_vmem2d(hist, _DEG_ROWS, HF)
    lane = lax.iota(jnp.int32, L)
    for k in range(_DEG_ROWS // L):
        rowidx[pl.ds(k * L, L)] = lane + k * L

    pltpu.sync_copy(src_hbm.at[s], srcb)
    pltpu.sync_copy(dst_hbm.at[s], dstb)
    ones = jnp.full((L,), 1.0, jnp.float32)

    def hist_body(j, _):
        for v in range(C // L):
            sv = srcb[j, pl.ds(v * L, L)]
            dv = dstb[j, pl.ds(v * L, L)]
            ne = sv != dv
            row = lax.shift_right_logical(sv, 7)
            col = lax.bitwise_and(sv, 127)
            for ln in range(L):
                m = lax.bitwise_and(ne, lane == ln)
                plsc.addupdate_scatter(hist, [row, col], ones, mask=m)
        return 0

    lax.fori_loop(0, NCH, hist_body, 0)

    plsc.subcore_barrier()
    pltpu.sync_copy(hist, deg_sh.at[rowidx], add=True)
    plsc.subcore_barrier()

    pltpu.sync_copy(deg_sh.at[pl.ds(s * rpt, rpt)], vbuf)
    magic = jnp.full((L,), 0x5F3759DF, jnp.int32)
    for r in range(rpt):
        for v in range(HF // L):
            d = vbuf[r, pl.ds(v * L, L)]
            di = plsc.bitcast(d, jnp.int32)
            y = plsc.bitcast(magic - lax.shift_right_logical(di, 1), jnp.float32)
            hd = 0.5 * d
            for _ in range(3):
                t = (hd * y) * y
                y = y * (1.5 - t)
            y = jnp.where(d > 0.0, y, 0.0)
            vout[pl.ds(r * HF + v * L, L)] = y
    @pl.when(c == 0)
    def _():
        pltpu.sync_copy(vout, dinv_hbm.at[pl.ds(s * rpt * HF, rpt * HF)])


@functools.partial(
    pl.kernel,
    out_type=[
        jax.ShapeDtypeStruct((NC, NPAD, HF), jnp.float32),
        jax.ShapeDtypeStruct((NC, NPAD, HF), jnp.float32),
        jax.ShapeDtypeStruct((X2ROWS, HF), jnp.float32),
    ],
    mesh=_MESH,
    compiler_params=pltpu.CompilerParams(needs_layout_passes=False),
    scratch_types=[
        [pltpu.VMEM((2, C), jnp.int32) for _ in range(NBUF)],
        [pltpu.VMEM((C,), jnp.int32) for _ in range(NBUF)],
        [pltpu.VMEM((C, HF), jnp.float32) for _ in range(NBUF)],
        pltpu.VMEM((16, HF), jnp.float32),
        pltpu.VMEM((C,), jnp.float32),
        [pltpu.SemaphoreType.DMA for _ in range(NBUF)],
        [pltpu.SemaphoreType.DMA for _ in range(NBUF)],
        pltpu.VMEM_SHARED((NPAD, HF), jnp.float32),
    ],
)
def _sc_prop2x(xf_hbm, ep_hbm, dinv_hbm, s1_hbm, s2_hbm, xf2_hbm,
               ebufs, gbufs, rbufs, zbuf, dinvq, gsems, ssems, acc):
    c = lax.axis_index("c")
    s = lax.axis_index("s")

    def zero_my_acc_rows():
        for k in range(NPT // 16):
            pltpu.sync_copy(zbuf, acc.at[pl.ds(s * NPT + k * 16, 16)])

    def stage_chunk(j, p, off, zrow):
        pltpu.sync_copy(ep_hbm.at[s, j], ebufs[p])
        for v in range(C // L):
            sv = ebufs[p][0, pl.ds(v * L, L)]
            dv = ebufs[p][1, pl.ds(v * L, L)]
            gbufs[p][pl.ds(v * L, L)] = jnp.where(sv != dv, sv, zrow) + off

    def run_pass(table_hbm, off, zrow):
        def issue_gather(j, p):
            pltpu.async_copy(table_hbm.at[gbufs[p]], rbufs[p], gsems[p])

        def wait_gather(j, p):
            pltpu.make_async_copy(table_hbm.at[gbufs[p]], rbufs[p], gsems[p]).wait()

        def issue_scatter(j, p):
            pltpu.async_copy(rbufs[p], acc.at[ebufs[p].at[1]], ssems[p], add=True)

        def wait_scatter(j, p):
            pltpu.make_async_copy(rbufs[p], acc.at[ebufs[p].at[1]], ssems[p]).wait()

        for p in range(NBUF):
            stage_chunk(p, p, off, zrow)
            issue_gather(p, p)

        def body(i, _):
            jb = i * NBUF
            for p in range(NBUF):
                j = jb + p
                wait_gather(j, p)
                issue_scatter(j, p)
            for p in range(NBUF):
                j = jb + p
                wait_scatter(j, p)
                stage_chunk(j + NBUF, p, off, zrow)
                issue_gather(j + NBUF, p)
            return 0

        ngroups = NCH // NBUF - 1
        lax.fori_loop(0, ngroups, body, 0)

        jb = ngroups * NBUF
        while jb < NCH:
            group = list(range(jb, min(jb + NBUF, NCH)))
            for j in group:
                p = j % NBUF
                wait_gather(j, p)
                issue_scatter(j, p)
            for j in group:
                p = j % NBUF
                wait_scatter(j, p)
                if j + NBUF < NCH:
                    stage_chunk(j + NBUF, p, off, zrow)
                    issue_gather(j + NBUF, p)
            jb += NBUF

    one16 = jnp.full((L,), 1, jnp.int32)
    _zero_vmem2d(zbuf, 16, HF)

    zero_my_acc_rows()
    plsc.subcore_barrier()
    run_pass(xf_hbm, c * N, jnp.int32(2 * N))
    plsc.subcore_barrier()

    for k in range(NPT // C):
        base = s * NPT + k * C
        pltpu.sync_copy(acc.at[pl.ds(base, C)], rbufs[0])
        pltpu.sync_copy(rbufs[0], s1_hbm.at[c, pl.ds(base, C)])
        pltpu.sync_copy(dinv_hbm.at[pl.ds(base, C)], dinvq)

        def row_body(r, _):
            dv = plsc.load_gather(dinvq, [one16 * r])
            gg = -dv * dv
            for v in range(HF // L):
                rbufs[1][r, pl.ds(v * L, L)] = gg * rbufs[0][r, pl.ds(v * L, L)]
            return 0

        lax.fori_loop(0, C, row_body, 0)
        pltpu.sync_copy(rbufs[1], xf2_hbm.at[pl.ds(c * NPAD + base, C)])

    zero_my_acc_rows()
    plsc.subcore_barrier()
    run_pass(xf2_hbm, c * NPAD, jnp.int32(N))
    plsc.subcore_barrier()

    pltpu.sync_copy(acc.at[pl.ds(s * NPT, NPT)], s2_hbm.at[c, pl.ds(s * NPT, NPT)])


_BR = 1000


def _b1_body(x_ref, d_ref, o_ref):
    i = pl.program_id(0)
    nb = N // _BR
    half = jnp.minimum(i // nb, 1)
    val = d_ref[...] * x_ref[:, pl.ds(half * HF, HF)]
    o_ref[...] = jnp.where(i < 2 * nb, val, 0.0)


def _tc_scale_flat(x, dinv2):
    grid = 2 * (N // _BR) + 1
    nb = N // _BR
    return pl.pallas_call(
        _b1_body,
        grid=(grid,),
        in_specs=[
            pl.BlockSpec((_BR, F), lambda i: (i % nb, 0)),
            pl.BlockSpec((_BR, 1), lambda i: (i % nb, 0)),
        ],
        out_specs=pl.BlockSpec((_BR, HF), lambda i: (i, 0)),
        out_shape=jax.ShapeDtypeStruct((XROWS, HF), jnp.float32),
    )(x, dinv2)


def _final_body(x_ref, s1a, s1b, s2a, s2b, d_ref, a0, a1, a2, bc, wl, bl, o_ref):
    d = d_ref[...]
    q1 = d * jnp.concatenate([s1a[0], s1b[0]], axis=1)
    q2 = d * jnp.concatenate([s2a[0], s2b[0]], axis=1)
    u = jnp.dot(x_ref[...], a0[...], preferred_element_type=jnp.float32)
    u += jnp.dot(q1, a1[...], preferred_element_type=jnp.float32)
    u += jnp.dot(q2, a2[...], preferred_element_type=jnp.float32)
    u += bc[...]
    h = jnp.maximum(u, 0.0)
    o_ref[...] = jnp.dot(h, wl[...], preferred_element_type=jnp.float32) + bl[...]


def _tc_final(x, s1, s2, dinv2, A0, A1, A2, bc, Wl, bl):
    nb = N // _BR
    wspec = pl.BlockSpec((F, F), lambda i: (0, 0))
    h0 = lambda i: (0, i, 0)
    h1 = lambda i: (1, i, 0)
    return pl.pallas_call(
        _final_body,
        grid=(nb,),
        in_specs=[
            pl.BlockSpec((_BR, F), lambda i: (i, 0)),
            pl.BlockSpec((1, _BR, HF), h0),
            pl.BlockSpec((1, _BR, HF), h1),
            pl.BlockSpec((1, _BR, HF), h0),
            pl.BlockSpec((1, _BR, HF), h1),
            pl.BlockSpec((_BR, 1), lambda i: (i, 0)),
            wspec, wspec, wspec,
            pl.BlockSpec((1, F), lambda i: (0, 0)),
            pl.BlockSpec((F, NCLASS), lambda i: (0, 0)),
            pl.BlockSpec((1, NCLASS), lambda i: (0, 0)),
        ],
        out_specs=pl.BlockSpec((_BR, NCLASS), lambda i: (i, 0)),
        out_shape=jax.ShapeDtypeStruct((N, NCLASS), jnp.float32),
    )(x, s1, s1, s2, s2, dinv2, A0, A1, A2, bc, Wl, bl)


def kernel(x, edge_index, W_cheb, b_cheb, W_lin, b_lin):
    ei = edge_index.astype(jnp.int32)
    srcb = ei[0].reshape(NS, NCH, C)
    dstb = ei[1].reshape(NS, NCH, C)
    epack = jnp.stack([srcb, dstb], axis=2)

    dinv1d = _sc_prep(srcb, dstb)
    dinv2 = dinv1d[:N].reshape(N, 1)

    x1f = _tc_scale_flat(x, dinv2)
    s1, s2, _ = _sc_prop2x(x1f, epack, dinv1d)

    A0 = W_cheb[0] - W_cheb[2]
    A1 = -W_cheb[1]
    A2 = -2.0 * W_cheb[2]
    return _tc_final(x, s1, s2, dinv2, A0, A1, A2,
                     b_cheb.reshape(1, F), W_lin, b_lin.reshape(1, NCLASS))


# speedup vs baseline: 12.0786x; 1.0004x over previous
import functools

import jax
import jax.numpy as jnp
from jax import lax
from jax.experimental import pallas as pl
from jax.experimental.pallas import tpu as pltpu
from jax.experimental.pallas import tpu_sc as plsc

N = 10000
E = 160000
F = 256
HF = 128
NCLASS = 40

NC = 2
NS = 16
L = 16

EPT = E // NS
C = 80
NCH = EPT // C
NPAD = 10240
NPT = NPAD // NS
XROWS = 2 * N + 16
X2ROWS = 2 * NPAD

_MESH = plsc.VectorSubcoreMesh(
    core_axis_name="c", subcore_axis_name="s", num_cores=NC, num_subcores=NS
)

_DEG_ROWS = 128
NBUF = 4


def _zero_vmem2d(ref, rows, cols):
    z = jnp.zeros((L,), ref.dtype)

    def body(j, _):
        for v in range(cols // L):
            ref[j, pl.ds(v * L, L)] = z
        return 0

    lax.fori_loop(0, rows, body, 0)


@functools.partial(
    pl.kernel,
    out_type=jax.ShapeDtypeStruct((_DEG_ROWS * HF,), jnp.float32),
    mesh=_MESH,
    compiler_params=pltpu.CompilerParams(needs_layout_passes=False),
    scratch_types=[
        pltpu.VMEM((NCH, C), jnp.int32),
        pltpu.VMEM((NCH, C), jnp.int32),
        pltpu.VMEM((_DEG_ROWS, HF), jnp.float32),
        pltpu.VMEM((_DEG_ROWS // NS, HF), jnp.float32),
        pltpu.VMEM((_DEG_ROWS,), jnp.int32),
        pltpu.VMEM((_DEG_ROWS // NS * HF,), jnp.float32),
        pltpu.VMEM_SHARED((_DEG_ROWS, HF), jnp.float32),
    ],
)
def _sc_prep(src_hbm, dst_hbm, dinv_hbm, srcb, dstb, hist, vbuf, rowidx, vout,
             deg_sh):
    c = lax.axis_index("c")
    s = lax.axis_index("s")
    rpt = _DEG_ROWS // NS

    _zero_vmem2d(vbuf, rpt, HF)
    pltpu.sync_copy(vbuf, deg_sh.at[pl.ds(s * rpt, rpt)])
    _zero_vmem2d(hist, _DEG_ROWS, HF)
    lane = lax.iota(jnp.int32, L)
    for k in range(_DEG_ROWS // L):
        rowidx[pl.ds(k * L, L)] = lane + k * L

    pltpu.sync_copy(src_hbm.at[s], srcb)
    pltpu.sync_copy(dst_hbm.at[s], dstb)
    ones = jnp.full((L,), 1.0, jnp.float32)

    def hist_body(j, _):
        for v in range(C // L):
            sv = srcb[j, pl.ds(v * L, L)]
            dv = dstb[j, pl.ds(v * L, L)]
            ne = sv != dv
            row = lax.shift_right_logical(sv, 7)
            col = lax.bitwise_and(sv, 127)
            for ln in range(L):
                m = lax.bitwise_and(ne, lane == ln)
                plsc.addupdate_scatter(hist, [row, col], ones, mask=m)
        return 0

    lax.fori_loop(0, NCH, hist_body, 0)

    plsc.subcore_barrier()
    pltpu.sync_copy(hist, deg_sh.at[rowidx], add=True)
    plsc.subcore_barrier()

    pltpu.sync_copy(deg_sh.at[pl.ds(s * rpt, rpt)], vbuf)
    magic = jnp.full((L,), 0x5F3759DF, jnp.int32)
    for r in range(rpt):
        for v in range(HF // L):
            d = vbuf[r, pl.ds(v * L, L)]
            di = plsc.bitcast(d, jnp.int32)
            y = plsc.bitcast(magic - lax.shift_right_logical(di, 1), jnp.float32)
            hd = 0.5 * d
            for _ in range(3):
                t = (hd * y) * y
                y = y * (1.5 - t)
            y = jnp.where(d > 0.0, y, 0.0)
            vout[pl.ds(r * HF + v * L, L)] = y
    @pl.when(c == 0)
    def _():
        pltpu.sync_copy(vout, dinv_hbm.at[pl.ds(s * rpt * HF, rpt * HF)])


@functools.partial(
    pl.kernel,
    out_type=[
        jax.ShapeDtypeStruct((NC, NPAD, HF), jnp.float32),
        jax.ShapeDtypeStruct((NC, NPAD, HF), jnp.float32),
        jax.ShapeDtypeStruct((X2ROWS, HF), jnp.float32),
    ],
    mesh=_MESH,
    compiler_params=pltpu.CompilerParams(needs_layout_passes=False),
    scratch_types=[
        pltpu.VMEM((2, C), jnp.int32),
        [pltpu.VMEM((C,), jnp.int32) for _ in range(2 * NBUF)],
        [pltpu.VMEM((C,), jnp.int32) for _ in range(2 * NBUF)],
        [pltpu.VMEM((C, HF), jnp.float32) for _ in range(NBUF)],
        pltpu.VMEM((16, HF), jnp.float32),
        pltpu.VMEM((C,), jnp.float32),
        [pltpu.SemaphoreType.DMA for _ in range(NBUF)],
        [pltpu.SemaphoreType.DMA for _ in range(NBUF)],
        pltpu.VMEM_SHARED((NPAD, HF), jnp.float32),
    ],
)
def _sc_prop2x(xf_hbm, ep_hbm, dinv_hbm, s1_hbm, s2_hbm, xf2_hbm,
               ebuf, gbufs, sbufs, rbufs, zbuf, dinvq, gsems, ssems, acc):
    c = lax.axis_index("c")
    s = lax.axis_index("s")

    def zero_my_acc_rows():
        for k in range(NPT // 16):
            pltpu.sync_copy(zbuf, acc.at[pl.ds(s * NPT + k * 16, 16)])

    def stage_chunk(j, q, off, zrow):
        pltpu.sync_copy(ep_hbm.at[s, j], ebuf)
        for v in range(C // L):
            sv = ebuf[0, pl.ds(v * L, L)]
            dv = ebuf[1, pl.ds(v * L, L)]
            gbufs[q][pl.ds(v * L, L)] = jnp.where(sv != dv, sv, zrow) + off
            sbufs[q][pl.ds(v * L, L)] = dv

    def run_pass(table_hbm, off, zrow):
        Q = 2 * NBUF

        def issue_gather(p, q):
            pltpu.async_copy(table_hbm.at[gbufs[q]], rbufs[p], gsems[p])

        def wait_gather(p, q):
            pltpu.make_async_copy(table_hbm.at[gbufs[q]], rbufs[p], gsems[p]).wait()

        def issue_scatter(p, q):
            pltpu.async_copy(rbufs[p], acc.at[sbufs[q]], ssems[p], add=True)

        def wait_scatter(p, q):
            pltpu.make_async_copy(rbufs[p], acc.at[sbufs[q]], ssems[p]).wait()

        for p in range(NBUF):
            stage_chunk(p, p, off, zrow)
            issue_gather(p, p)

        def body(i, _):
            jb = i * Q
            for g in range(2):
                for p in range(NBUF):
                    q = 4 * g + p
                    wait_gather(p, q)
                    issue_scatter(p, q)
                for p in range(NBUF):
                    q = 4 * g + p
                    qn = (q + NBUF) % Q
                    stage_chunk(jb + 4 * g + p + NBUF, qn, off, zrow)
                    wait_scatter(p, q)
                    issue_gather(p, qn)
            return 0

        ngroups = NCH // Q
        lax.fori_loop(0, ngroups, body, 0)

        jb = ngroups * Q
        while jb < NCH:
            group = list(range(jb, min(jb + NBUF, NCH)))
            for j in group:
                p, q = j % NBUF, j % Q
                wait_gather(p, q)
                issue_scatter(p, q)
            for j in group:
                p, q = j % NBUF, j % Q
                if j + NBUF < NCH:
                    qn = (j + NBUF) % Q
                    stage_chunk(j + NBUF, qn, off, zrow)
                    wait_scatter(p, q)
                    issue_gather(p, qn)
                else:
                    wait_scatter(p, q)
            jb += NBUF

    one16 = jnp.full((L,), 1, jnp.int32)
    _zero_vmem2d(zbuf, 16, HF)

    zero_my_acc_rows()
    plsc.subcore_barrier()
    run_pass(xf_hbm, c * N, jnp.int32(2 * N))
    plsc.subcore_barrier()

    for k in range(NPT // C):
        base = s * NPT + k * C
        pltpu.sync_copy(acc.at[pl.ds(base, C)], rbufs[0])
        pltpu.sync_copy(rbufs[0], s1_hbm.at[c, pl.ds(base, C)])
        pltpu.sync_copy(dinv_hbm.at[pl.ds(base, C)], dinvq)

        def row_body(r, _):
            dv = plsc.load_gather(dinvq, [one16 * r])
            gg = -dv * dv
            for v in range(HF // L):
                rbufs[1][r, pl.ds(v * L, L)] = gg * rbufs[0][r, pl.ds(v * L, L)]
            return 0

        lax.fori_loop(0, C, row_body, 0)
        pltpu.sync_copy(rbufs[1], xf2_hbm.at[pl.ds(c * NPAD + base, C)])

    zero_my_acc_rows()
    plsc.subcore_barrier()
    run_pass(xf2_hbm, c * NPAD, jnp.int32(N))
    plsc.subcore_barrier()

    pltpu.sync_copy(acc.at[pl.ds(s * NPT, NPT)], s2_hbm.at[c, pl.ds(s * NPT, NPT)])


_BR = 1000


def _b1_body(x_ref, d_ref, o_ref):
    i = pl.program_id(0)
    nb = N // _BR
    half = jnp.minimum(i // nb, 1)
    val = d_ref[...] * x_ref[:, pl.ds(half * HF, HF)]
    o_ref[...] = jnp.where(i < 2 * nb, val, 0.0)


def _tc_scale_flat(x, dinv2):
    grid = 2 * (N // _BR) + 1
    nb = N // _BR
    return pl.pallas_call(
        _b1_body,
        grid=(grid,),
        in_specs=[
            pl.BlockSpec((_BR, F), lambda i: (i % nb, 0)),
            pl.BlockSpec((_BR, 1), lambda i: (i % nb, 0)),
        ],
        out_specs=pl.BlockSpec((_BR, HF), lambda i: (i, 0)),
        out_shape=jax.ShapeDtypeStruct((XROWS, HF), jnp.float32),
    )(x, dinv2)


def _final_body(x_ref, s1a, s1b, s2a, s2b, d_ref, a0, a1, a2, bc, wl, bl, o_ref):
    d = d_ref[...]
    q1 = d * jnp.concatenate([s1a[0], s1b[0]], axis=1)
    q2 = d * jnp.concatenate([s2a[0], s2b[0]], axis=1)
    u = jnp.dot(x_ref[...], a0[...], preferred_element_type=jnp.float32)
    u += jnp.dot(q1, a1[...], preferred_element_type=jnp.float32)
    u += jnp.dot(q2, a2[...], preferred_element_type=jnp.float32)
    u += bc[...]
    h = jnp.maximum(u, 0.0)
    o_ref[...] = jnp.dot(h, wl[...], preferred_element_type=jnp.float32) + bl[...]


def _tc_final(x, s1, s2, dinv2, A0, A1, A2, bc, Wl, bl):
    nb = N // _BR
    wspec = pl.BlockSpec((F, F), lambda i: (0, 0))
    h0 = lambda i: (0, i, 0)
    h1 = lambda i: (1, i, 0)
    return pl.pallas_call(
        _final_body,
        grid=(nb,),
        in_specs=[
            pl.BlockSpec((_BR, F), lambda i: (i, 0)),
            pl.BlockSpec((1, _BR, HF), h0),
            pl.BlockSpec((1, _BR, HF), h1),
            pl.BlockSpec((1, _BR, HF), h0),
            pl.BlockSpec((1, _BR, HF), h1),
            pl.BlockSpec((_BR, 1), lambda i: (i, 0)),
            wspec, wspec, wspec,
            pl.BlockSpec((1, F), lambda i: (0, 0)),
            pl.BlockSpec((F, NCLASS), lambda i: (0, 0)),
            pl.BlockSpec((1, NCLASS), lambda i: (0, 0)),
        ],
        out_specs=pl.BlockSpec((_BR, NCLASS), lambda i: (i, 0)),
        out_shape=jax.ShapeDtypeStruct((N, NCLASS), jnp.float32),
    )(x, s1, s1, s2, s2, dinv2, A0, A1, A2, bc, Wl, bl)


def kernel(x, edge_index, W_cheb, b_cheb, W_lin, b_lin):
    ei = edge_index.astype(jnp.int32)
    srcb = ei[0].reshape(NS, NCH, C)
    dstb = ei[1].reshape(NS, NCH, C)
    epack = jnp.stack([srcb, dstb], axis=2)

    dinv1d = _sc_prep(srcb, dstb)
    dinv2 = dinv1d[:N].reshape(N, 1)

    x1f = _tc_scale_flat(x, dinv2)
    s1, s2, _ = _sc_prop2x(x1f, epack, dinv1d)

    A0 = W_cheb[0] - W_cheb[2]
    A1 = -W_cheb[1]
    A2 = -2.0 * W_cheb[2]
    return _tc_final(x, s1, s2, dinv2, A0, A1, A2,
                     b_cheb.reshape(1, F), W_lin, b_lin.reshape(1, NCLASS))


# speedup vs baseline: 12.0928x; 1.0012x over previous
import functools

import jax
import jax.numpy as jnp
from jax import lax
from jax.experimental import pallas as pl
from jax.experimental.pallas import tpu as pltpu
from jax.experimental.pallas import tpu_sc as plsc

N = 10000
E = 160000
F = 256
HF = 128
NCLASS = 40

NC = 2
NS = 16
L = 16

EPT = E // NS
C = 80
NCH = EPT // C
NPAD = 10240
NPT = NPAD // NS
XROWS = 2 * N + 16
X2ROWS = 2 * NPAD

_MESH = plsc.VectorSubcoreMesh(
    core_axis_name="c", subcore_axis_name="s", num_cores=NC, num_subcores=NS
)

_DEG_ROWS = 128
NBUF = 4


def _zero_vmem2d(ref, rows, cols):
    z = jnp.zeros((L,), ref.dtype)

    def body(j, _):
        for v in range(cols // L):
            ref[j, pl.ds(v * L, L)] = z
        return 0

    lax.fori_loop(0, rows, body, 0)


@functools.partial(
    pl.kernel,
    out_type=jax.ShapeDtypeStruct((_DEG_ROWS * HF,), jnp.float32),
    mesh=_MESH,
    compiler_params=pltpu.CompilerParams(needs_layout_passes=False),
    scratch_types=[
        pltpu.VMEM((NCH, C), jnp.int32),
        pltpu.VMEM((NCH, C), jnp.int32),
        pltpu.VMEM((_DEG_ROWS, HF), jnp.float32),
        pltpu.VMEM((_DEG_ROWS // NS, HF), jnp.float32),
        pltpu.VMEM((_DEG_ROWS,), jnp.int32),
        pltpu.VMEM((_DEG_ROWS // NS * HF,), jnp.float32),
        pltpu.VMEM_SHARED((_DEG_ROWS, HF), jnp.float32),
    ],
)
def _sc_prep(src_hbm, dst_hbm, dinv_hbm, srcb, dstb, hist, vbuf, rowidx, vout,
             deg_sh):
    c = lax.axis_index("c")
    s = lax.axis_index("s")
    rpt = _DEG_ROWS // NS

    _zero_vmem2d(vbuf, rpt, HF)
    pltpu.sync_copy(vbuf, deg_sh.at[pl.ds(s * rpt, rpt)])
    _zero_vmem2d(hist, _DEG_ROWS, HF)
    lane = lax.iota(jnp.int32, L)
    for k in range(_DEG_ROWS // L):
        rowidx[pl.ds(k * L, L)] = lane + k * L

    pltpu.sync_copy(src_hbm.at[s], srcb)
    pltpu.sync_copy(dst_hbm.at[s], dstb)
    ones = jnp.full((L,), 1.0, jnp.float32)

    def hist_body(j, _):
        for v in range(C // L):
            sv = srcb[j, pl.ds(v * L, L)]
            dv = dstb[j, pl.ds(v * L, L)]
            ne = sv != dv
            row = lax.shift_right_logical(sv, 7)
            col = lax.bitwise_and(sv, 127)
            for ln in range(L):
                m = lax.bitwise_and(ne, lane == ln)
                plsc.addupdate_scatter(hist, [row, col], ones, mask=m)
        return 0

    lax.fori_loop(0, NCH, hist_body, 0)

    plsc.subcore_barrier()
    pltpu.sync_copy(hist, deg_sh.at[rowidx], add=True)
    plsc.subcore_barrier()

    pltpu.sync_copy(deg_sh.at[pl.ds(s * rpt, rpt)], vbuf)
    magic = jnp.full((L,), 0x5F3759DF, jnp.int32)
    for r in range(rpt):
        for v in range(HF // L):
            d = vbuf[r, pl.ds(v * L, L)]
            di = plsc.bitcast(d, jnp.int32)
            y = plsc.bitcast(magic - lax.shift_right_logical(di, 1), jnp.float32)
            hd = 0.5 * d
            for _ in range(3):
                t = (hd * y) * y
                y = y * (1.5 - t)
            y = jnp.where(d > 0.0, y, 0.0)
            vout[pl.ds(r * HF + v * L, L)] = y
    @pl.when(c == 0)
    def _():
        pltpu.sync_copy(vout, dinv_hbm.at[pl.ds(s * rpt * HF, rpt * HF)])


@functools.partial(
    pl.kernel,
    out_type=[
        jax.ShapeDtypeStruct((NC, NPAD, HF), jnp.float32),
        jax.ShapeDtypeStruct((NC, NPAD, HF), jnp.float32),
        jax.ShapeDtypeStruct((X2ROWS, HF), jnp.float32),
    ],
    mesh=_MESH,
    compiler_params=pltpu.CompilerParams(needs_layout_passes=False),
    scratch_types=[
        pltpu.VMEM((2, C), jnp.int32),
        pltpu.VMEM((2 * NBUF, C), jnp.int32),
        pltpu.VMEM((2 * NBUF, C), jnp.int32),
        [pltpu.VMEM((C, HF), jnp.float32) for _ in range(NBUF)],
        pltpu.VMEM((16, HF), jnp.float32),
        pltpu.VMEM((C,), jnp.float32),
        [pltpu.SemaphoreType.DMA for _ in range(NBUF)],
        [pltpu.SemaphoreType.DMA for _ in range(NBUF)],
        pltpu.VMEM_SHARED((NPAD, HF), jnp.float32),
    ],
)
def _sc_prop2x(xf_hbm, ep_hbm, dinv_hbm, s1_hbm, s2_hbm, xf2_hbm,
               ebuf, gbuf, sbuf, rbufs, zbuf, dinvq, gsems, ssems, acc):
    c = lax.axis_index("c")
    s = lax.axis_index("s")

    def zero_my_acc_rows():
        for k in range(NPT // 16):
            pltpu.sync_copy(zbuf, acc.at[pl.ds(s * NPT + k * 16, 16)])

    def stage_chunk(j, q, off, zrow):
        pltpu.sync_copy(ep_hbm.at[s, j], ebuf)
        for v in range(C // L):
            sv = ebuf[0, pl.ds(v * L, L)]
            dv = ebuf[1, pl.ds(v * L, L)]
            gbuf[q, pl.ds(v * L, L)] = jnp.where(sv != dv, sv, zrow) + off
            sbuf[q, pl.ds(v * L, L)] = dv

    def run_pass(table_hbm, off, zrow):
        Q = 2 * NBUF

        def issue_gather(p, q):
            pltpu.async_copy(table_hbm.at[gbuf.at[q]], rbufs[p], gsems[p])

        def wait_gather(p, q):
            pltpu.make_async_copy(table_hbm.at[gbuf.at[q]], rbufs[p], gsems[p]).wait()

        def issue_scatter(p, q):
            pltpu.async_copy(rbufs[p], acc.at[sbuf.at[q]], ssems[p], add=True)

        def wait_scatter(p, q):
            pltpu.make_async_copy(rbufs[p], acc.at[sbuf.at[q]], ssems[p]).wait()

        for p in range(NBUF):
            stage_chunk(p, p, off, zrow)
            issue_gather(p, p)

        def body(i, _):
            jb = i * Q
            for g in range(2):
                for p in range(NBUF):
                    q = 4 * g + p
                    wait_gather(p, q)
                    issue_scatter(p, q)
                for p in range(NBUF):
                    q = 4 * g + p
                    qn = (q + NBUF) % Q
                    stage_chunk(jb + 4 * g + p + NBUF, qn, off, zrow)
                    wait_scatter(p, q)
                    issue_gather(p, qn)
            return 0

        ngroups = NCH // Q
        lax.fori_loop(0, ngroups, body, 0)

        jb = ngroups * Q
        while jb < NCH:
            group = list(range(jb, min(jb + NBUF, NCH)))
            for j in group:
                p, q = j % NBUF, j % Q
                wait_gather(p, q)
                issue_scatter(p, q)
            for j in group:
                p, q = j % NBUF, j % Q
                if j + NBUF < NCH:
                    qn = (j + NBUF) % Q
                    stage_chunk(j + NBUF, qn, off, zrow)
                    wait_scatter(p, q)
                    issue_gather(p, qn)
                else:
                    wait_scatter(p, q)
            jb += NBUF

    one16 = jnp.full((L,), 1, jnp.int32)
    _zero_vmem2d(zbuf, 16, HF)

    zero_my_acc_rows()
    plsc.subcore_barrier()
    run_pass(xf_hbm, c * N, jnp.int32(2 * N))
    plsc.subcore_barrier()

    for k in range(NPT // C):
        base = s * NPT + k * C
        pltpu.sync_copy(acc.at[pl.ds(base, C)], rbufs[0])
        pltpu.sync_copy(rbufs[0], s1_hbm.at[c, pl.ds(base, C)])
        pltpu.sync_copy(dinv_hbm.at[pl.ds(base, C)], dinvq)

        def row_body(r, _):
            dv = plsc.load_gather(dinvq, [one16 * r])
            gg = -dv * dv
            for v in range(HF // L):
                rbufs[1][r, pl.ds(v * L, L)] = gg * rbufs[0][r, pl.ds(v * L, L)]
            return 0

        lax.fori_loop(0, C, row_body, 0)
        pltpu.sync_copy(rbufs[1], xf2_hbm.at[pl.ds(c * NPAD + base, C)])

    zero_my_acc_rows()
    plsc.subcore_barrier()
    run_pass(xf2_hbm, c * NPAD, jnp.int32(N))
    plsc.subcore_barrier()

    pltpu.sync_copy(acc.at[pl.ds(s * NPT, NPT)], s2_hbm.at[c, pl.ds(s * NPT, NPT)])


_BR = 1000


def _b1_body(x_ref, d_ref, o_ref):
    i = pl.program_id(0)
    nb = N // _BR
    half = jnp.minimum(i // nb, 1)
    val = d_ref[...] * x_ref[:, pl.ds(half * HF, HF)]
    o_ref[...] = jnp.where(i < 2 * nb, val, 0.0)


def _tc_scale_flat(x, dinv2):
    grid = 2 * (N // _BR) + 1
    nb = N // _BR
    return pl.pallas_call(
        _b1_body,
        grid=(grid,),
        in_specs=[
            pl.BlockSpec((_BR, F), lambda i: (i % nb, 0)),
            pl.BlockSpec((_BR, 1), lambda i: (i % nb, 0)),
        ],
        out_specs=pl.BlockSpec((_BR, HF), lambda i: (i, 0)),
        out_shape=jax.ShapeDtypeStruct((XROWS, HF), jnp.float32),
    )(x, dinv2)


def _final_body(x_ref, s1a, s1b, s2a, s2b, d_ref, a0, a1, a2, bc, wl, bl, o_ref):
    d = d_ref[...]
    q1 = d * jnp.concatenate([s1a[0], s1b[0]], axis=1)
    q2 = d * jnp.concatenate([s2a[0], s2b[0]], axis=1)
    u = jnp.dot(x_ref[...], a0[...], preferred_element_type=jnp.float32)
    u += jnp.dot(q1, a1[...], preferred_element_type=jnp.float32)
    u += jnp.dot(q2, a2[...], preferred_element_type=jnp.float32)
    u += bc[...]
    h = jnp.maximum(u, 0.0)
    o_ref[...] = jnp.dot(h, wl[...], preferred_element_type=jnp.float32) + bl[...]


def _tc_final(x, s1, s2, dinv2, A0, A1, A2, bc, Wl, bl):
    nb = N // _BR
    wspec = pl.BlockSpec((F, F), lambda i: (0, 0))
    h0 = lambda i: (0, i, 0)
    h1 = lambda i: (1, i, 0)
    return pl.pallas_call(
        _final_body,
        grid=(nb,),
        in_specs=[
            pl.BlockSpec((_BR, F), lambda i: (i, 0)),
            pl.BlockSpec((1, _BR, HF), h0),
            pl.BlockSpec((1, _BR, HF), h1),
            pl.BlockSpec((1, _BR, HF), h0),
            pl.BlockSpec((1, _BR, HF), h1),
            pl.BlockSpec((_BR, 1), lambda i: (i, 0)),
            wspec, wspec, wspec,
            pl.BlockSpec((1, F), lambda i: (0, 0)),
            pl.BlockSpec((F, NCLASS), lambda i: (0, 0)),
            pl.BlockSpec((1, NCLASS), lambda i: (0, 0)),
        ],
        out_specs=pl.BlockSpec((_BR, NCLASS), lambda i: (i, 0)),
        out_shape=jax.ShapeDtypeStruct((N, NCLASS), jnp.float32),
    )(x, s1, s1, s2, s2, dinv2, A0, A1, A2, bc, Wl, bl)


def kernel(x, edge_index, W_cheb, b_cheb, W_lin, b_lin):
    ei = edge_index.astype(jnp.int32)
    srcb = ei[0].reshape(NS, NCH, C)
    dstb = ei[1].reshape(NS, NCH, C)
    epack = jnp.stack([srcb, dstb], axis=2)

    dinv1d = _sc_prep(srcb, dstb)
    dinv2 = dinv1d[:N].reshape(N, 1)

    x1f = _tc_scale_flat(x, dinv2)
    s1, s2, _ = _sc_prop2x(x1f, epack, dinv1d)

    A0 = W_cheb[0] - W_cheb[2]
    A1 = -W_cheb[1]
    A2 = -2.0 * W_cheb[2]
    return _tc_final(x, s1, s2, dinv2, A0, A1, A2,
                     b_cheb.reshape(1, F), W_lin, b_lin.reshape(1, NCLASS))


# speedup vs baseline: 12.8368x; 1.0615x over previous
import functools

import jax
import jax.numpy as jnp
from jax import lax
from jax.experimental import pallas as pl
from jax.experimental.pallas import tpu as pltpu
from jax.experimental.pallas import tpu_sc as plsc

N = 10000
E = 160000
F = 256
HF = 128
NCLASS = 40

NC = 2
NS = 16
L = 16

EPT = E // NS
C = 80
NCH = EPT // C
NPAD = 10240
NPT = NPAD // NS
XROWS = 2 * N + 16
X2ROWS = 2 * NPAD

_MESH = plsc.VectorSubcoreMesh(
    core_axis_name="c", subcore_axis_name="s", num_cores=NC, num_subcores=NS
)

_DEG_ROWS = 128
NBUF = 4


def _zero_vmem2d(ref, rows, cols):
    z = jnp.zeros((L,), ref.dtype)

    def body(j, _):
        for v in range(cols // L):
            ref[j, pl.ds(v * L, L)] = z
        return 0

    lax.fori_loop(0, rows, body, 0)


@functools.partial(
    pl.kernel,
    out_type=jax.ShapeDtypeStruct((_DEG_ROWS * HF,), jnp.float32),
    mesh=_MESH,
    compiler_params=pltpu.CompilerParams(needs_layout_passes=False),
    scratch_types=[
        pltpu.VMEM((NCH, C), jnp.int32),
        pltpu.VMEM((NCH, C), jnp.int32),
        pltpu.VMEM((_DEG_ROWS, HF), jnp.float32),
        pltpu.VMEM((_DEG_ROWS // NS, HF), jnp.float32),
        pltpu.VMEM((_DEG_ROWS,), jnp.int32),
        pltpu.VMEM((_DEG_ROWS // NS * HF,), jnp.float32),
        pltpu.VMEM_SHARED((_DEG_ROWS, HF), jnp.float32),
    ],
)
def _sc_prep(src_hbm, dst_hbm, dinv_hbm, srcb, dstb, hist, vbuf, rowidx, vout,
             deg_sh):
    c = lax.axis_index("c")
    s = lax.axis_index("s")
    rpt = _DEG_ROWS // NS

    _zero_vmem2d(vbuf, rpt, HF)
    pltpu.sync_copy(vbuf, deg_sh.at[pl.ds(s * rpt, rpt)])
    _zero_vmem2d(hist, _DEG_ROWS, HF)
    lane = lax.iota(jnp.int32, L)
    for k in range(_DEG_ROWS // L):
        rowidx[pl.ds(k * L, L)] = lane + k * L

    pltpu.sync_copy(src_hbm.at[s], srcb)
    pltpu.sync_copy(dst_hbm.at[s], dstb)
    ones = jnp.full((L,), 1.0, jnp.float32)

    def hist_body(j, _):
        for v in range(C // L):
            sv = srcb[j, pl.ds(v * L, L)]
            dv = dstb[j, pl.ds(v * L, L)]
            ne = sv != dv
            row = lax.shift_right_logical(sv, 7)
            col = lax.bitwise_and(sv, 127)
            for ln in range(L):
                m = lax.bitwise_and(ne, lane == ln)
                plsc.addupdate_scatter(hist, [row, col], ones, mask=m)
        return 0

    lax.fori_loop(0, NCH, hist_body, 0)

    plsc.subcore_barrier()
    pltpu.sync_copy(hist, deg_sh.at[rowidx], add=True)
    plsc.subcore_barrier()

    pltpu.sync_copy(deg_sh.at[pl.ds(s * rpt, rpt)], vbuf)
    magic = jnp.full((L,), 0x5F3759DF, jnp.int32)
    for r in range(rpt):
        for v in range(HF // L):
            d = vbuf[r, pl.ds(v * L, L)]
            di = plsc.bitcast(d, jnp.int32)
            y = plsc.bitcast(magic - lax.shift_right_logical(di, 1), jnp.float32)
            hd = 0.5 * d
            for _ in range(3):
                t = (hd * y) * y
                y = y * (1.5 - t)
            y = jnp.where(d > 0.0, y, 0.0)
            vout[pl.ds(r * HF + v * L, L)] = y
    @pl.when(c == 0)
    def _():
        pltpu.sync_copy(vout, dinv_hbm.at[pl.ds(s * rpt * HF, rpt * HF)])


@functools.partial(
    pl.kernel,
    out_type=[
        jax.ShapeDtypeStruct((NC, NPAD, HF), jnp.float32),
        jax.ShapeDtypeStruct((NC, NPAD, HF), jnp.float32),
        jax.ShapeDtypeStruct((X2ROWS, HF), jnp.float32),
    ],
    mesh=_MESH,
    compiler_params=pltpu.CompilerParams(needs_layout_passes=False),
    scratch_types=[
        pltpu.VMEM((2, 2 * NBUF, C), jnp.int32),
        pltpu.VMEM((2 * NBUF, C), jnp.int32),
        pltpu.VMEM((2 * NBUF, C), jnp.int32),
        [pltpu.VMEM((C, HF), jnp.float32) for _ in range(NBUF)],
        pltpu.VMEM((16, HF), jnp.float32),
        pltpu.VMEM((C,), jnp.float32),
        [pltpu.SemaphoreType.DMA for _ in range(NBUF)],
        [pltpu.SemaphoreType.DMA for _ in range(NBUF)],
        pltpu.SemaphoreType.DMA,
        pltpu.VMEM_SHARED((NPAD, HF), jnp.float32),
    ],
)
def _sc_prop2x(xf_hbm, ep_hbm, dinv_hbm, s1_hbm, s2_hbm, xf2_hbm,
               ebatch, gbuf, sbuf, rbufs, zbuf, dinvq, gsems, ssems, zsem, acc):
    c = lax.axis_index("c")
    s = lax.axis_index("s")

    def zero_my_acc_rows():
        for g in range(2):
            for k in range(20 * g, 20 * (g + 1)):
                pltpu.async_copy(zbuf, acc.at[pl.ds(s * NPT + k * 16, 16)], zsem)
            for k in range(20 * g, 20 * (g + 1)):
                pltpu.make_async_copy(zbuf, acc.at[pl.ds(s * NPT + k * 16, 16)],
                                      zsem).wait()

    def load_batch(b):
        pltpu.sync_copy(ep_hbm.at[s, b], ebatch)

    def stage_chunk(local, q, off, zrow):
        for v in range(C // L):
            sv = ebatch[0, local, pl.ds(v * L, L)]
            dv = ebatch[1, local, pl.ds(v * L, L)]
            gbuf[q, pl.ds(v * L, L)] = jnp.where(sv != dv, sv, zrow) + off
            sbuf[q, pl.ds(v * L, L)] = dv

    def run_pass(table_hbm, off, zrow):
        Q = 2 * NBUF

        def issue_gather(p, q):
            pltpu.async_copy(table_hbm.at[gbuf.at[q]], rbufs[p], gsems[p])

        def wait_gather(p, q):
            pltpu.make_async_copy(table_hbm.at[gbuf.at[q]], rbufs[p], gsems[p]).wait()

        def issue_scatter(p, q):
            pltpu.async_copy(rbufs[p], acc.at[sbuf.at[q]], ssems[p], add=True)

        def wait_scatter(p, q):
            pltpu.make_async_copy(rbufs[p], acc.at[sbuf.at[q]], ssems[p]).wait()

        load_batch(0)
        for p in range(NBUF):
            stage_chunk(p, p, off, zrow)
            issue_gather(p, p)

        def body(i, _):
            for p in range(NBUF):
                wait_gather(p, p)
                issue_scatter(p, p)
            for p in range(NBUF):
                stage_chunk(NBUF + p, NBUF + p, off, zrow)
                wait_scatter(p, p)
                issue_gather(p, NBUF + p)
            load_batch(i + 1)
            for p in range(NBUF):
                wait_gather(p, NBUF + p)
                issue_scatter(p, NBUF + p)
            for p in range(NBUF):
                stage_chunk(p, p, off, zrow)
                wait_scatter(p, NBUF + p)
                issue_gather(p, p)
            return 0

        ngroups = NCH // Q
        lax.fori_loop(0, ngroups, body, 0)

        for p in range(NBUF):
            wait_gather(p, p)
            issue_scatter(p, p)
        stage_chunk(NBUF, NBUF, off, zrow)
        wait_scatter(0, 0)
        issue_gather(0, NBUF)
        for p in range(1, NBUF):
            wait_scatter(p, p)
        wait_gather(0, NBUF)
        issue_scatter(0, NBUF)
        wait_scatter(0, NBUF)

    one16 = jnp.full((L,), 1, jnp.int32)
    _zero_vmem2d(zbuf, 16, HF)

    zero_my_acc_rows()
    plsc.subcore_barrier()
    run_pass(xf_hbm, c * N, jnp.int32(2 * N))
    plsc.subcore_barrier()

    for k in range(NPT // C):
        base = s * NPT + k * C
        pltpu.sync_copy(acc.at[pl.ds(base, C)], rbufs[0])
        pltpu.sync_copy(rbufs[0], s1_hbm.at[c, pl.ds(base, C)])
        pltpu.sync_copy(dinv_hbm.at[pl.ds(base, C)], dinvq)

        def row_body(r, _):
            dv = plsc.load_gather(dinvq, [one16 * r])
            gg = -dv * dv
            for v in range(HF // L):
                rbufs[1][r, pl.ds(v * L, L)] = gg * rbufs[0][r, pl.ds(v * L, L)]
            return 0

        lax.fori_loop(0, C, row_body, 0)
        pltpu.sync_copy(rbufs[1], xf2_hbm.at[pl.ds(c * NPAD + base, C)])

    zero_my_acc_rows()
    plsc.subcore_barrier()
    run_pass(xf2_hbm, c * NPAD, jnp.int32(N))
    plsc.subcore_barrier()

    pltpu.sync_copy(acc.at[pl.ds(s * NPT, NPT)], s2_hbm.at[c, pl.ds(s * NPT, NPT)])


_BR = 1000


def _b1_body(x_ref, d_ref, o_ref):
    i = pl.program_id(0)
    nb = N // _BR
    half = jnp.minimum(i // nb, 1)
    val = d_ref[...] * x_ref[:, pl.ds(half * HF, HF)]
    o_ref[...] = jnp.where(i < 2 * nb, val, 0.0)


def _tc_scale_flat(x, dinv2):
    grid = 2 * (N // _BR) + 1
    nb = N // _BR
    return pl.pallas_call(
        _b1_body,
        grid=(grid,),
        in_specs=[
            pl.BlockSpec((_BR, F), lambda i: (i % nb, 0)),
            pl.BlockSpec((_BR, 1), lambda i: (i % nb, 0)),
        ],
        out_specs=pl.BlockSpec((_BR, HF), lambda i: (i, 0)),
        out_shape=jax.ShapeDtypeStruct((XROWS, HF), jnp.float32),
    )(x, dinv2)


def _final_body(x_ref, s1a, s1b, s2a, s2b, d_ref, a0, a1, a2, bc, wl, bl, o_ref):
    d = d_ref[...]
    q1 = d * jnp.concatenate([s1a[0], s1b[0]], axis=1)
    q2 = d * jnp.concatenate([s2a[0], s2b[0]], axis=1)
    u = jnp.dot(x_ref[...], a0[...], preferred_element_type=jnp.float32)
    u += jnp.dot(q1, a1[...], preferred_element_type=jnp.float32)
    u += jnp.dot(q2, a2[...], preferred_element_type=jnp.float32)
    u += bc[...]
    h = jnp.maximum(u, 0.0)
    o_ref[...] = jnp.dot(h, wl[...], preferred_element_type=jnp.float32) + bl[...]


def _tc_final(x, s1, s2, dinv2, A0, A1, A2, bc, Wl, bl):
    nb = N // _BR
    wspec = pl.BlockSpec((F, F), lambda i: (0, 0))
    h0 = lambda i: (0, i, 0)
    h1 = lambda i: (1, i, 0)
    return pl.pallas_call(
        _final_body,
        grid=(nb,),
        in_specs=[
            pl.BlockSpec((_BR, F), lambda i: (i, 0)),
            pl.BlockSpec((1, _BR, HF), h0),
            pl.BlockSpec((1, _BR, HF), h1),
            pl.BlockSpec((1, _BR, HF), h0),
            pl.BlockSpec((1, _BR, HF), h1),
            pl.BlockSpec((_BR, 1), lambda i: (i, 0)),
            wspec, wspec, wspec,
            pl.BlockSpec((1, F), lambda i: (0, 0)),
            pl.BlockSpec((F, NCLASS), lambda i: (0, 0)),
            pl.BlockSpec((1, NCLASS), lambda i: (0, 0)),
        ],
        out_specs=pl.BlockSpec((_BR, NCLASS), lambda i: (i, 0)),
        out_shape=jax.ShapeDtypeStruct((N, NCLASS), jnp.float32),
    )(x, s1, s1, s2, s2, dinv2, A0, A1, A2, bc, Wl, bl)


def kernel(x, edge_index, W_cheb, b_cheb, W_lin, b_lin):
    ei = edge_index.astype(jnp.int32)
    srcb = ei[0].reshape(NS, NCH, C)
    dstb = ei[1].reshape(NS, NCH, C)
    pad = jnp.zeros((NS, 2 * NBUF * C * ((NCH + 2 * NBUF - 1) // (2 * NBUF)) - EPT),
                    jnp.int32)
    srcp = jnp.concatenate([ei[0].reshape(NS, EPT), pad], axis=1)
    dstp = jnp.concatenate([ei[1].reshape(NS, EPT), pad], axis=1)
    nb = srcp.shape[1] // (2 * NBUF * C)
    epack = jnp.stack([srcp.reshape(NS, nb, 2 * NBUF, C),
                       dstp.reshape(NS, nb, 2 * NBUF, C)], axis=2)

    dinv1d = _sc_prep(srcb, dstb)
    dinv2 = dinv1d[:N].reshape(N, 1)

    x1f = _tc_scale_flat(x, dinv2)
    s1, s2, _ = _sc_prop2x(x1f, epack, dinv1d)

    A0 = W_cheb[0] - W_cheb[2]
    A1 = -W_cheb[1]
    A2 = -2.0 * W_cheb[2]
    return _tc_final(x, s1, s2, dinv2, A0, A1, A2,
                     b_cheb.reshape(1, F), W_lin, b_lin.reshape(1, NCLASS))
